# Initial kernel scaffold; baseline (speedup 1.0000x reference)
import jax
import jax.numpy as jnp
from jax.experimental import pallas as pl


def kernel(x, codebook0, codebook1, codebook2, codebook3, rotateMatrix):
    raise NotImplementedError("write your pallas kernel here")


# speedup vs baseline: 1.5590x; 1.5590x over previous
import jax
import jax.numpy as jnp
from jax.experimental import pallas as pl

_M = 4
_K = 8192
_D = 256
_SPLIT = _D // _M
_N = 8192
_TN = 128


def _main_body(x_ref, r_ref, c_ref, idx_ref, stats_ref):
    xt = x_ref[...]
    rm = r_ref[0]
    split = jnp.dot(xt, rm, preferred_element_type=jnp.float32)
    c = c_ref[0]
    cc = jnp.sum(c * c, axis=1)
    dots = jax.lax.dot_general(split, c, (((1,), (1,)), ((), ())),
                               preferred_element_type=jnp.float32)
    s = 2.0 * dots - cc[None, :]
    mx = jnp.max(s, axis=1, keepdims=True)
    iota = jax.lax.broadcasted_iota(jnp.int32, (_TN, _K), 1)
    idx = jnp.min(jnp.where(s >= mx, iota, _K), axis=1, keepdims=True)
    p = jnp.exp(s - mx)
    den = jnp.sum(p, axis=1, keepdims=True)
    soft = jax.lax.dot_general(p, c, (((1,), (0,)), ((), ())),
                               preferred_element_type=jnp.float32) / den
    oh = (iota == idx).astype(jnp.float32)
    hard = jax.lax.dot_general(oh, c, (((1,), (0,)), ((), ())),
                               preferred_element_type=jnp.float32)
    d1 = split - soft
    d2 = split - hard
    d3 = soft - hard
    lane = jax.lax.broadcasted_iota(jnp.int32, (1, 128), 1)
    v = jnp.where(lane == 0, jnp.sum(d1 * d1),
                  jnp.where(lane == 1, jnp.sum(d2 * d2),
                            jnp.where(lane == 2, jnp.sum(d3 * d3), 0.0)))
    stats_ref[...] = v.reshape(1, 1, 1, 128)
    idx_ref[...] = idx.reshape(1, _TN, 1)


def _reg_body(r_ref, o_ref):
    r = r_ref[...]
    rrt = jax.lax.dot_general(r, r, (((1,), (1,)), ((), ())),
                              preferred_element_type=jnp.float32)
    i0 = jax.lax.broadcasted_iota(jnp.int32, (_D, _D), 0)
    i1 = jax.lax.broadcasted_iota(jnp.int32, (_D, _D), 1)
    d = rrt - (i0 == i1).astype(jnp.float32)
    o_ref[...] = (jnp.sum(d * d) / float(_D * _D)).reshape(1, 1)


def kernel(x, codebook0, codebook1, codebook2, codebook3, rotateMatrix):
    cs = jnp.stack([codebook0, codebook1, codebook2, codebook3], axis=0)
    rs = rotateMatrix.reshape(_D, _M, _SPLIT).transpose(1, 0, 2)
    nt = _N // _TN
    idx_out, stats = pl.pallas_call(
        _main_body,
        grid=(_M, nt),
        in_specs=[
            pl.BlockSpec((_TN, _D), lambda m, n: (n, 0)),
            pl.BlockSpec((1, _D, _SPLIT), lambda m, n: (m, 0, 0)),
            pl.BlockSpec((1, _K, _SPLIT), lambda m, n: (m, 0, 0)),
        ],
        out_specs=[
            pl.BlockSpec((1, _TN, 1), lambda m, n: (m, n, 0)),
            pl.BlockSpec((1, 1, 1, 128), lambda m, n: (m, n, 0, 0)),
        ],
        out_shape=[
            jax.ShapeDtypeStruct((_M, _N, 1), jnp.int32),
            jax.ShapeDtypeStruct((_M, nt, 1, 128), jnp.float32),
        ],
    )(x, rs, cs)
    reg = pl.pallas_call(
        _reg_body,
        out_shape=jax.ShapeDtypeStruct((1, 1), jnp.float32),
    )(rotateMatrix)
    hardCodes = idx_out.reshape(_M, _N).T
    s = jnp.sum(stats.reshape(_M * nt, 128), axis=0)
    denom = float(_N * _SPLIT)
    loss = (0.1 * s[0] + s[1] + 0.1 * s[2]) / denom + 0.01 * reg[0, 0]
    return hardCodes, loss


# speedup vs baseline: 1.5798x; 1.0133x over previous
import jax
import jax.numpy as jnp
from jax.experimental import pallas as pl

_M = 4
_K = 8192
_D = 256
_SPLIT = _D // _M
_N = 8192
_TN = 128


def _main_body(x_ref, r_ref, c_ref, idx_ref, stats_ref):
    xt = x_ref[...]
    rm = r_ref[0]
    split = jnp.dot(xt, rm, preferred_element_type=jnp.float32)
    c = c_ref[0]
    cc = jnp.sum(c * c, axis=1)
    dots2 = jax.lax.dot_general(2.0 * split, c, (((1,), (1,)), ((), ())),
                                preferred_element_type=jnp.float32)
    s = dots2 - cc[None, :]
    mx = jnp.max(s, axis=1, keepdims=True)
    iota = jax.lax.broadcasted_iota(jnp.int32, (_TN, _K), 1)
    idx = jnp.min(jnp.where(s >= mx, iota, _K), axis=1, keepdims=True)
    p = jnp.exp(s - mx)
    den = jnp.sum(p, axis=1, keepdims=True)
    soft = jax.lax.dot_general(p, c, (((1,), (0,)), ((), ())),
                               preferred_element_type=jnp.float32) / den
    oh = (iota == idx).astype(jnp.float32)
    hard = jax.lax.dot_general(oh, c, (((1,), (0,)), ((), ())),
                               preferred_element_type=jnp.float32)
    d1 = split - soft
    d2 = split - hard
    d3 = soft - hard
    lane = jax.lax.broadcasted_iota(jnp.int32, (1, 128), 1)
    v = jnp.where(lane == 0, jnp.sum(d1 * d1),
                  jnp.where(lane == 1, jnp.sum(d2 * d2),
                            jnp.where(lane == 2, jnp.sum(d3 * d3), 0.0)))
    stats_ref[...] = v.reshape(1, 1, 1, 128)
    idx_ref[...] = idx.reshape(1, _TN, 1)


def _reg_body(r_ref, o_ref):
    r = r_ref[...]
    rrt = jax.lax.dot_general(r, r, (((1,), (1,)), ((), ())),
                              preferred_element_type=jnp.float32)
    i0 = jax.lax.broadcasted_iota(jnp.int32, (_D, _D), 0)
    i1 = jax.lax.broadcasted_iota(jnp.int32, (_D, _D), 1)
    d = rrt - (i0 == i1).astype(jnp.float32)
    o_ref[...] = (jnp.sum(d * d) / float(_D * _D)).reshape(1, 1)


def kernel(x, codebook0, codebook1, codebook2, codebook3, rotateMatrix):
    cs = jnp.stack([codebook0, codebook1, codebook2, codebook3], axis=0)
    rs = rotateMatrix.reshape(_D, _M, _SPLIT).transpose(1, 0, 2)
    nt = _N // _TN
    idx_out, stats = pl.pallas_call(
        _main_body,
        grid=(_M, nt),
        in_specs=[
            pl.BlockSpec((_TN, _D), lambda m, n: (n, 0)),
            pl.BlockSpec((1, _D, _SPLIT), lambda m, n: (m, 0, 0)),
            pl.BlockSpec((1, _K, _SPLIT), lambda m, n: (m, 0, 0)),
        ],
        out_specs=[
            pl.BlockSpec((1, _TN, 1), lambda m, n: (m, n, 0)),
            pl.BlockSpec((1, 1, 1, 128), lambda m, n: (m, n, 0, 0)),
        ],
        out_shape=[
            jax.ShapeDtypeStruct((_M, _N, 1), jnp.int32),
            jax.ShapeDtypeStruct((_M, nt, 1, 128), jnp.float32),
        ],
    )(x, rs, cs)
    reg = pl.pallas_call(
        _reg_body,
        out_shape=jax.ShapeDtypeStruct((1, 1), jnp.float32),
    )(rotateMatrix)
    hardCodes = idx_out.reshape(_M, _N).T
    s = jnp.sum(stats.reshape(_M * nt, 128), axis=0)
    denom = float(_N * _SPLIT)
    loss = (0.1 * s[0] + s[1] + 0.1 * s[2]) / denom + 0.01 * reg[0, 0]
    return hardCodes, loss


# speedup vs baseline: 1.8191x; 1.1515x over previous
import jax
import jax.numpy as jnp
from jax.experimental import pallas as pl

_M = 4
_K = 8192
_D = 256
_SPLIT = _D // _M
_N = 8192
_TN = 256


def _main_body(x_ref, r_ref, c_ref, idx_ref, stats_ref):
    xt = x_ref[...]
    rm = r_ref[0]
    split = jnp.dot(xt, rm, preferred_element_type=jnp.float32)
    c = c_ref[0]
    cc = jnp.sum(c * c, axis=1)
    dots2 = jax.lax.dot_general(2.0 * split, c, (((1,), (1,)), ((), ())),
                                preferred_element_type=jnp.float32)
    s = dots2 - cc[None, :]
    mx = jnp.max(s, axis=1, keepdims=True)
    iota = jax.lax.broadcasted_iota(jnp.int32, (_TN, _K), 1)
    idx = jnp.min(jnp.where(s >= mx, iota, _K), axis=1, keepdims=True)
    p = jnp.exp(s - mx)
    den = jnp.sum(p, axis=1, keepdims=True)
    soft = jax.lax.dot_general(p, c, (((1,), (0,)), ((), ())),
                               preferred_element_type=jnp.float32) / den
    oh = (iota == idx).astype(jnp.float32)
    hard = jax.lax.dot_general(oh, c, (((1,), (0,)), ((), ())),
                               preferred_element_type=jnp.float32)
    d1 = split - soft
    d2 = split - hard
    d3 = soft - hard
    lane = jax.lax.broadcasted_iota(jnp.int32, (1, 128), 1)
    v = jnp.where(lane == 0, jnp.sum(d1 * d1),
                  jnp.where(lane == 1, jnp.sum(d2 * d2),
                            jnp.where(lane == 2, jnp.sum(d3 * d3), 0.0)))
    stats_ref[...] = v.reshape(1, 1, 1, 128)
    idx_ref[...] = idx.reshape(1, _TN, 1)


def _reg_body(r_ref, o_ref):
    r = r_ref[...]
    rrt = jax.lax.dot_general(r, r, (((1,), (1,)), ((), ())),
                              preferred_element_type=jnp.float32)
    i0 = jax.lax.broadcasted_iota(jnp.int32, (_D, _D), 0)
    i1 = jax.lax.broadcasted_iota(jnp.int32, (_D, _D), 1)
    d = rrt - (i0 == i1).astype(jnp.float32)
    o_ref[...] = (jnp.sum(d * d) / float(_D * _D)).reshape(1, 1)


def kernel(x, codebook0, codebook1, codebook2, codebook3, rotateMatrix):
    cs = jnp.stack([codebook0, codebook1, codebook2, codebook3], axis=0)
    rs = rotateMatrix.reshape(_D, _M, _SPLIT).transpose(1, 0, 2)
    nt = _N // _TN
    idx_out, stats = pl.pallas_call(
        _main_body,
        grid=(_M, nt),
        in_specs=[
            pl.BlockSpec((_TN, _D), lambda m, n: (n, 0)),
            pl.BlockSpec((1, _D, _SPLIT), lambda m, n: (m, 0, 0)),
            pl.BlockSpec((1, _K, _SPLIT), lambda m, n: (m, 0, 0)),
        ],
        out_specs=[
            pl.BlockSpec((1, _TN, 1), lambda m, n: (m, n, 0)),
            pl.BlockSpec((1, 1, 1, 128), lambda m, n: (m, n, 0, 0)),
        ],
        out_shape=[
            jax.ShapeDtypeStruct((_M, _N, 1), jnp.int32),
            jax.ShapeDtypeStruct((_M, nt, 1, 128), jnp.float32),
        ],
    )(x, rs, cs)
    reg = pl.pallas_call(
        _reg_body,
        out_shape=jax.ShapeDtypeStruct((1, 1), jnp.float32),
    )(rotateMatrix)
    hardCodes = idx_out.reshape(_M, _N).T
    s = jnp.sum(stats.reshape(_M * nt, 128), axis=0)
    denom = float(_N * _SPLIT)
    loss = (0.1 * s[0] + s[1] + 0.1 * s[2]) / denom + 0.01 * reg[0, 0]
    return hardCodes, loss


# speedup vs baseline: 1.9554x; 1.0749x over previous
import jax
import jax.numpy as jnp
from jax.experimental import pallas as pl

_M = 4
_K = 8192
_D = 256
_SPLIT = _D // _M
_N = 8192
_TN = 512


def _main_body(x_ref, r_ref, c_ref, idx_ref, stats_ref):
    xt = x_ref[...]
    rm = r_ref[0]
    split = jnp.dot(xt, rm, preferred_element_type=jnp.float32)
    c = c_ref[0]
    cc = jnp.sum(c * c, axis=1)
    dots2 = jax.lax.dot_general(2.0 * split, c, (((1,), (1,)), ((), ())),
                                preferred_element_type=jnp.float32)
    s = dots2 - cc[None, :]
    mx = jnp.max(s, axis=1, keepdims=True)
    iota = jax.lax.broadcasted_iota(jnp.int32, (_TN, _K), 1)
    idx = jnp.min(jnp.where(s >= mx, iota, _K), axis=1, keepdims=True)
    p = jnp.exp(s - mx)
    den = jnp.sum(p, axis=1, keepdims=True)
    soft = jax.lax.dot_general(p, c, (((1,), (0,)), ((), ())),
                               preferred_element_type=jnp.float32) / den
    oh = (iota == idx).astype(jnp.float32)
    hard = jax.lax.dot_general(oh, c, (((1,), (0,)), ((), ())),
                               preferred_element_type=jnp.float32)
    d1 = split - soft
    d2 = split - hard
    d3 = soft - hard
    lane = jax.lax.broadcasted_iota(jnp.int32, (1, 128), 1)
    v = jnp.where(lane == 0, jnp.sum(d1 * d1),
                  jnp.where(lane == 1, jnp.sum(d2 * d2),
                            jnp.where(lane == 2, jnp.sum(d3 * d3), 0.0)))
    stats_ref[...] = v.reshape(1, 1, 1, 128)
    idx_ref[...] = idx.reshape(1, _TN, 1)


def _reg_body(r_ref, o_ref):
    r = r_ref[...]
    rrt = jax.lax.dot_general(r, r, (((1,), (1,)), ((), ())),
                              preferred_element_type=jnp.float32)
    i0 = jax.lax.broadcasted_iota(jnp.int32, (_D, _D), 0)
    i1 = jax.lax.broadcasted_iota(jnp.int32, (_D, _D), 1)
    d = rrt - (i0 == i1).astype(jnp.float32)
    o_ref[...] = (jnp.sum(d * d) / float(_D * _D)).reshape(1, 1)


def kernel(x, codebook0, codebook1, codebook2, codebook3, rotateMatrix):
    cs = jnp.stack([codebook0, codebook1, codebook2, codebook3], axis=0)
    rs = rotateMatrix.reshape(_D, _M, _SPLIT).transpose(1, 0, 2)
    nt = _N // _TN
    idx_out, stats = pl.pallas_call(
        _main_body,
        grid=(_M, nt),
        in_specs=[
            pl.BlockSpec((_TN, _D), lambda m, n: (n, 0)),
            pl.BlockSpec((1, _D, _SPLIT), lambda m, n: (m, 0, 0)),
            pl.BlockSpec((1, _K, _SPLIT), lambda m, n: (m, 0, 0)),
        ],
        out_specs=[
            pl.BlockSpec((1, _TN, 1), lambda m, n: (m, n, 0)),
            pl.BlockSpec((1, 1, 1, 128), lambda m, n: (m, n, 0, 0)),
        ],
        out_shape=[
            jax.ShapeDtypeStruct((_M, _N, 1), jnp.int32),
            jax.ShapeDtypeStruct((_M, nt, 1, 128), jnp.float32),
        ],
    )(x, rs, cs)
    reg = pl.pallas_call(
        _reg_body,
        out_shape=jax.ShapeDtypeStruct((1, 1), jnp.float32),
    )(rotateMatrix)
    hardCodes = idx_out.reshape(_M, _N).T
    s = jnp.sum(stats.reshape(_M * nt, 128), axis=0)
    denom = float(_N * _SPLIT)
    loss = (0.1 * s[0] + s[1] + 0.1 * s[2]) / denom + 0.01 * reg[0, 0]
    return hardCodes, loss


# speedup vs baseline: 2.0672x; 1.0571x over previous
import jax
import jax.numpy as jnp
from jax.experimental import pallas as pl

_M = 4
_K = 8192
_D = 256
_SPLIT = _D // _M
_N = 8192
_TN = 1024


def _main_body(x_ref, r_ref, c_ref, idx_ref, stats_ref):
    xt = x_ref[...]
    rm = r_ref[0]
    split = jnp.dot(xt, rm, preferred_element_type=jnp.float32)
    c = c_ref[0]
    cc = jnp.sum(c * c, axis=1)
    dots2 = jax.lax.dot_general(2.0 * split, c, (((1,), (1,)), ((), ())),
                                preferred_element_type=jnp.float32)
    s = dots2 - cc[None, :]
    mx = jnp.max(s, axis=1, keepdims=True)
    iota = jax.lax.broadcasted_iota(jnp.int32, (_TN, _K), 1)
    idx = jnp.min(jnp.where(s >= mx, iota, _K), axis=1, keepdims=True)
    p = jnp.exp(s - mx)
    den = jnp.sum(p, axis=1, keepdims=True)
    soft = jax.lax.dot_general(p, c, (((1,), (0,)), ((), ())),
                               preferred_element_type=jnp.float32) / den
    oh = (iota == idx).astype(jnp.float32)
    hard = jax.lax.dot_general(oh, c, (((1,), (0,)), ((), ())),
                               preferred_element_type=jnp.float32)
    d1 = split - soft
    d2 = split - hard
    d3 = soft - hard
    lane = jax.lax.broadcasted_iota(jnp.int32, (1, 128), 1)
    v = jnp.where(lane == 0, jnp.sum(d1 * d1),
                  jnp.where(lane == 1, jnp.sum(d2 * d2),
                            jnp.where(lane == 2, jnp.sum(d3 * d3), 0.0)))
    stats_ref[...] = v.reshape(1, 1, 1, 128)
    idx_ref[...] = idx.reshape(1, _TN, 1)


def _reg_body(r_ref, o_ref):
    r = r_ref[...]
    rrt = jax.lax.dot_general(r, r, (((1,), (1,)), ((), ())),
                              preferred_element_type=jnp.float32)
    i0 = jax.lax.broadcasted_iota(jnp.int32, (_D, _D), 0)
    i1 = jax.lax.broadcasted_iota(jnp.int32, (_D, _D), 1)
    d = rrt - (i0 == i1).astype(jnp.float32)
    o_ref[...] = (jnp.sum(d * d) / float(_D * _D)).reshape(1, 1)


def kernel(x, codebook0, codebook1, codebook2, codebook3, rotateMatrix):
    cs = jnp.stack([codebook0, codebook1, codebook2, codebook3], axis=0)
    rs = rotateMatrix.reshape(_D, _M, _SPLIT).transpose(1, 0, 2)
    nt = _N // _TN
    idx_out, stats = pl.pallas_call(
        _main_body,
        grid=(_M, nt),
        in_specs=[
            pl.BlockSpec((_TN, _D), lambda m, n: (n, 0)),
            pl.BlockSpec((1, _D, _SPLIT), lambda m, n: (m, 0, 0)),
            pl.BlockSpec((1, _K, _SPLIT), lambda m, n: (m, 0, 0)),
        ],
        out_specs=[
            pl.BlockSpec((1, _TN, 1), lambda m, n: (m, n, 0)),
            pl.BlockSpec((1, 1, 1, 128), lambda m, n: (m, n, 0, 0)),
        ],
        out_shape=[
            jax.ShapeDtypeStruct((_M, _N, 1), jnp.int32),
            jax.ShapeDtypeStruct((_M, nt, 1, 128), jnp.float32),
        ],
    )(x, rs, cs)
    reg = pl.pallas_call(
        _reg_body,
        out_shape=jax.ShapeDtypeStruct((1, 1), jnp.float32),
    )(rotateMatrix)
    hardCodes = idx_out.reshape(_M, _N).T
    s = jnp.sum(stats.reshape(_M * nt, 128), axis=0)
    denom = float(_N * _SPLIT)
    loss = (0.1 * s[0] + s[1] + 0.1 * s[2]) / denom + 0.01 * reg[0, 0]
    return hardCodes, loss


# speedup vs baseline: 2.4088x; 1.1653x over previous
import jax
import jax.numpy as jnp
from jax.experimental import pallas as pl

_M = 4
_K = 8192
_D = 256
_SPLIT = _D // _M
_N = 8192
_TN = 1024


def _main_body(x_ref, r_ref, c_ref, idx_ref, stats_ref):
    xt = x_ref[...]
    rm = r_ref[0]
    split = jnp.dot(xt, rm, preferred_element_type=jnp.float32)
    c = c_ref[0]
    cc = jnp.sum(c * c, axis=1)
    dots2 = jax.lax.dot_general(2.0 * split, c, (((1,), (1,)), ((), ())),
                                preferred_element_type=jnp.float32)
    s = dots2 - cc[None, :]
    mx = jnp.max(s, axis=1, keepdims=True)
    ge = s >= mx
    iota = jax.lax.broadcasted_iota(jnp.int32, (_TN, _K), 1)
    idx = jnp.min(jnp.where(ge, iota, _K), axis=1, keepdims=True)
    p = jnp.exp(s - mx)
    oh = jnp.where(ge, 1.0, 0.0)
    lane65 = jax.lax.broadcasted_iota(jnp.int32, (_K, _SPLIT + 1), 1)
    c_ext = jnp.where(lane65 < _SPLIT, jnp.pad(c, ((0, 0), (0, 1))), 1.0)
    soft_den = jax.lax.dot_general(p, c_ext, (((1,), (0,)), ((), ())),
                                   preferred_element_type=jnp.float32)
    den = soft_den[:, _SPLIT:_SPLIT + 1]
    soft = soft_den[:, :_SPLIT] / den
    hard = jax.lax.dot_general(oh, c, (((1,), (0,)), ((), ())),
                               preferred_element_type=jnp.float32)
    d1 = split - soft
    d2 = split - hard
    d3 = soft - hard
    lane = jax.lax.broadcasted_iota(jnp.int32, (1, 128), 1)
    v = jnp.where(lane == 0, jnp.sum(d1 * d1),
                  jnp.where(lane == 1, jnp.sum(d2 * d2),
                            jnp.where(lane == 2, jnp.sum(d3 * d3), 0.0)))
    stats_ref[...] = v.reshape(1, 1, 1, 128)
    idx_ref[...] = idx.reshape(1, _TN, 1)


def _reg_body(r_ref, o_ref):
    r = r_ref[...]
    rrt = jax.lax.dot_general(r, r, (((1,), (1,)), ((), ())),
                              preferred_element_type=jnp.float32)
    i0 = jax.lax.broadcasted_iota(jnp.int32, (_D, _D), 0)
    i1 = jax.lax.broadcasted_iota(jnp.int32, (_D, _D), 1)
    d = rrt - (i0 == i1).astype(jnp.float32)
    o_ref[...] = (jnp.sum(d * d) / float(_D * _D)).reshape(1, 1)


def kernel(x, codebook0, codebook1, codebook2, codebook3, rotateMatrix):
    cs = jnp.stack([codebook0, codebook1, codebook2, codebook3], axis=0)
    rs = rotateMatrix.reshape(_D, _M, _SPLIT).transpose(1, 0, 2)
    nt = _N // _TN
    idx_out, stats = pl.pallas_call(
        _main_body,
        grid=(_M, nt),
        in_specs=[
            pl.BlockSpec((_TN, _D), lambda m, n: (n, 0)),
            pl.BlockSpec((1, _D, _SPLIT), lambda m, n: (m, 0, 0)),
            pl.BlockSpec((1, _K, _SPLIT), lambda m, n: (m, 0, 0)),
        ],
        out_specs=[
            pl.BlockSpec((1, _TN, 1), lambda m, n: (m, n, 0)),
            pl.BlockSpec((1, 1, 1, 128), lambda m, n: (m, n, 0, 0)),
        ],
        out_shape=[
            jax.ShapeDtypeStruct((_M, _N, 1), jnp.int32),
            jax.ShapeDtypeStruct((_M, nt, 1, 128), jnp.float32),
        ],
    )(x, rs, cs)
    reg = pl.pallas_call(
        _reg_body,
        out_shape=jax.ShapeDtypeStruct((1, 1), jnp.float32),
    )(rotateMatrix)
    hardCodes = idx_out.reshape(_M, _N).T
    s = jnp.sum(stats.reshape(_M * nt, 128), axis=0)
    denom = float(_N * _SPLIT)
    loss = (0.1 * s[0] + s[1] + 0.1 * s[2]) / denom + 0.01 * reg[0, 0]
    return hardCodes, loss
